# baseline (device time: 21251 ns/iter reference)
import jax
import jax.numpy as jnp
from jax import lax
from jax.experimental import pallas as pl
from jax.experimental.pallas import tpu as pltpu

P = 16


def kernel(x, w_mat):
    m_per, k = x.shape
    _, n = w_mat.shape
    n_per = n // P
    m = m_per * P

    def body(x_ref, w_ref, out_ref, y_ref, y_bf, recv_buf,
             send_sems, recv_sems, ready_sems):
        my = lax.axis_index("i")

        for d in range(1, P):
            pl.semaphore_signal(
                ready_sems.at[my], inc=1,
                device_id=((my + d) % P,),
                device_id_type=pl.DeviceIdType.MESH,
            )

        barrier_sem = pltpu.get_barrier_semaphore()
        for nbr in [(my + 1) % P, (my - 1) % P]:
            pl.semaphore_signal(
                barrier_sem, inc=1,
                device_id=(nbr,), device_id_type=pl.DeviceIdType.MESH,
            )
        pl.semaphore_wait(barrier_sem, 2)

        y_ref[...] = jnp.dot(
            x_ref[...], w_ref[...], preferred_element_type=jnp.float32
        )
        y_bf[...] = y_ref[...].astype(jnp.bfloat16)

        out_ref[pl.ds(my * m_per, m_per), :] = y_ref[:, pl.ds(my * n_per, n_per)]

        for d in range(1, P):
            dst = (my + d) % P
            pl.semaphore_wait(ready_sems.at[dst], 1)
            rdma = pltpu.make_async_remote_copy(
                src_ref=y_bf.at[:, pl.ds(dst * n_per, n_per)],
                dst_ref=recv_buf.at[pl.ds(my * m_per, m_per), :],
                send_sem=send_sems.at[d - 1],
                recv_sem=recv_sems.at[my],
                device_id=(dst,),
                device_id_type=pl.DeviceIdType.MESH,
            )
            rdma.start()

        for d in range(1, P):
            src = (my - d) % P
            recv = pltpu.make_async_remote_copy(
                src_ref=y_bf.at[:, pl.ds(0, n_per)],
                dst_ref=recv_buf.at[pl.ds(src * m_per, m_per), :],
                send_sem=send_sems.at[0],
                recv_sem=recv_sems.at[src],
                device_id=(my,),
                device_id_type=pl.DeviceIdType.MESH,
            )
            recv.wait_recv()
            out_ref[pl.ds(src * m_per, m_per), :] = recv_buf[
                pl.ds(src * m_per, m_per), :
            ].astype(jnp.float32)

        for d in range(1, P):
            send = pltpu.make_async_remote_copy(
                src_ref=y_bf.at[:, pl.ds(0, n_per)],
                dst_ref=recv_buf.at[pl.ds(0, m_per), :],
                send_sem=send_sems.at[d - 1],
                recv_sem=recv_sems.at[my],
                device_id=(my,),
                device_id_type=pl.DeviceIdType.MESH,
            )
            send.wait_send()

    x = x.astype(jnp.bfloat16)
    w_mat = w_mat.astype(jnp.bfloat16)
    return pl.pallas_call(
        body,
        out_shape=jax.ShapeDtypeStruct((m, n_per), jnp.float32),
        in_specs=[
            pl.BlockSpec(memory_space=pltpu.VMEM),
            pl.BlockSpec(memory_space=pltpu.VMEM),
        ],
        out_specs=pl.BlockSpec(memory_space=pltpu.VMEM),
        scratch_shapes=[
            pltpu.VMEM((m_per, n), jnp.float32),
            pltpu.VMEM((m_per, n), jnp.bfloat16),
            pltpu.VMEM((m, n_per), jnp.bfloat16),
            pltpu.SemaphoreType.DMA((P - 1,)),
            pltpu.SemaphoreType.DMA((P,)),
            pltpu.SemaphoreType.REGULAR((P,)),
        ],
        compiler_params=pltpu.CompilerParams(collective_id=0),
    )(x, w_mat)
